# Initial kernel scaffold; baseline (speedup 1.0000x reference)
import jax
import jax.numpy as jnp
from jax.experimental import pallas as pl


def kernel(token_ids, embeddings):
    raise NotImplementedError("write your pallas kernel here")


# speedup vs baseline: 1.8618x; 1.8618x over previous
import functools

import jax
import jax.numpy as jnp
from jax import lax
from jax.experimental import pallas as pl
from jax.experimental.pallas import tpu as pltpu
from jax.experimental.pallas import tpu_sc as plsc


def _make_gather(N, V, D, num_cores, num_subcores, CH=640):
    NW = num_cores * num_subcores
    per_w = N // NW
    n_ch = per_w // CH
    n_outer = n_ch // 2

    mesh = plsc.VectorSubcoreMesh(core_axis_name="c", subcore_axis_name="s")

    @functools.partial(
        pl.kernel,
        mesh=mesh,
        out_type=jax.ShapeDtypeStruct((N, D), jnp.float32),
        scratch_types=[
            pltpu.VMEM((n_ch, CH), jnp.int32),
            pltpu.VMEM((CH, D), jnp.float32),
            pltpu.VMEM((CH, D), jnp.float32),
            pltpu.SemaphoreType.DMA,
            pltpu.SemaphoreType.DMA,
            pltpu.SemaphoreType.DMA,
            pltpu.SemaphoreType.DMA,
        ],
        compiler_params=pltpu.CompilerParams(use_tc_tiling_on_sc=False),
    )
    def gather_kernel(idx_hbm, table_hbm, out_hbm, idx_v, buf0, buf1, g0, g1, w0, w1):
        wid = lax.axis_index("s") * num_cores + lax.axis_index("c")
        cbase = wid * n_ch

        pltpu.sync_copy(idx_hbm.at[pl.ds(cbase, n_ch)], idx_v)

        def gather(i, buf, sem):
            pltpu.async_copy(table_hbm.at[idx_v.at[i]], buf, sem)

        def gather_wait(i, buf, sem):
            pltpu.make_async_copy(table_hbm.at[idx_v.at[i]], buf, sem).wait()

        def writeback(i, buf, sem):
            pltpu.async_copy(buf, out_hbm.at[pl.ds((cbase + i) * CH, CH)], sem)

        def writeback_wait(i, buf, sem):
            pltpu.make_async_copy(
                buf, out_hbm.at[pl.ds((cbase + i) * CH, CH)], sem
            ).wait()

        gather(0, buf0, g0)

        def body(j, carry):
            i = 2 * j
            gather(i + 1, buf1, g1)
            gather_wait(i, buf0, g0)
            writeback(i, buf0, w0)
            gather_wait(i + 1, buf1, g1)
            writeback(i + 1, buf1, w1)
            writeback_wait(i, buf0, w0)
            gather(i + 2, buf0, g0)
            writeback_wait(i + 1, buf1, w1)
            return carry

        lax.fori_loop(0, n_outer - 1, body, 0)

        i = n_ch - 2
        gather(i + 1, buf1, g1)
        gather_wait(i, buf0, g0)
        writeback(i, buf0, w0)
        gather_wait(i + 1, buf1, g1)
        writeback(i + 1, buf1, w1)
        writeback_wait(i, buf0, w0)
        writeback_wait(i + 1, buf1, w1)

    return gather_kernel


def kernel(token_ids, embeddings):
    B, H = token_ids.shape
    V, D = embeddings.shape
    flat = token_ids.reshape(-1).astype(jnp.int32)
    N = flat.shape[0]
    info = plsc.get_sparse_core_info()
    NW = info.num_cores * info.num_subcores
    CH = 640
    idx2d = flat.reshape(N // CH, CH)
    out = _make_gather(N, V, D, info.num_cores, info.num_subcores, CH)(
        idx2d, embeddings
    )
    return out.reshape(B, H, D)


# speedup vs baseline: 1.8739x; 1.0065x over previous
import functools

import jax
import jax.numpy as jnp
from jax import lax
from jax.experimental import pallas as pl
from jax.experimental.pallas import tpu as pltpu
from jax.experimental.pallas import tpu_sc as plsc

_CH = 128
_NBUF = 10
_P = 8


def _make_gather(N, V, D, num_cores, num_subcores):
    NW = num_cores * num_subcores
    per_w = N // NW
    CH, NBUF, P = _CH, _NBUF, _P
    n_ch = per_w // CH
    n_outer = n_ch // NBUF
    assert n_ch % NBUF == 0 and n_outer >= 3

    mesh = plsc.VectorSubcoreMesh(core_axis_name="c", subcore_axis_name="s")

    scratch = [
        pltpu.VMEM((n_ch, CH), jnp.int32),
        pltpu.VMEM((NBUF, CH, D), jnp.float32),
    ]
    scratch += [pltpu.SemaphoreType.DMA] * (2 * NBUF)

    @functools.partial(
        pl.kernel,
        mesh=mesh,
        out_type=jax.ShapeDtypeStruct((N, D), jnp.float32),
        scratch_types=scratch,
        compiler_params=pltpu.CompilerParams(use_tc_tiling_on_sc=False),
    )
    def gather_kernel(idx_hbm, table_hbm, out_hbm, idx_v, bufs, *sems):
        gsem = sems[:NBUF]
        wsem = sems[NBUF:]
        wid = lax.axis_index("s") * num_cores + lax.axis_index("c")
        cbase = wid * n_ch

        pltpu.sync_copy(idx_hbm.at[pl.ds(cbase, n_ch)], idx_v)

        def gather(i, b):
            pltpu.async_copy(table_hbm.at[idx_v.at[i]], bufs.at[b], gsem[b])

        def gather_wait(i, b):
            pltpu.make_async_copy(
                table_hbm.at[idx_v.at[i]], bufs.at[b], gsem[b]
            ).wait()

        def writeback(i, b):
            pltpu.async_copy(
                bufs.at[b], out_hbm.at[pl.ds((cbase + i) * CH, CH)], wsem[b]
            )

        def writeback_wait(i, b):
            pltpu.make_async_copy(
                bufs.at[b], out_hbm.at[pl.ds((cbase + i) * CH, CH)], wsem[b]
            ).wait()

        def step(i, b, wb_wait, fire):
            gather_wait(i, b)
            writeback(i, b)
            if wb_wait:
                writeback_wait(i - 2, (b - 2) % NBUF)
            if fire:
                gather(i + P, (b + P) % NBUF)

        for b in range(P):
            gather(b, b)
        for b in range(NBUF):
            step(b, b, wb_wait=(b >= 2), fire=True)

        def body(g, carry):
            i0 = g * NBUF
            for b in range(NBUF):
                step(i0 + b, b, wb_wait=True, fire=True)
            return carry

        lax.fori_loop(1, n_outer - 1, body, 0)

        i0 = (n_outer - 1) * NBUF
        for b in range(NBUF):
            step(i0 + b, b, wb_wait=True, fire=(i0 + b + P < n_ch))
        writeback_wait(n_ch - 2, (NBUF - 2) % NBUF)
        writeback_wait(n_ch - 1, NBUF - 1)

    return gather_kernel


def kernel(token_ids, embeddings):
    B, H = token_ids.shape
    V, D = embeddings.shape
    flat = token_ids.reshape(-1).astype(jnp.int32)
    N = flat.shape[0]
    info = plsc.get_sparse_core_info()
    idx2d = flat.reshape(N // _CH, _CH)
    out = _make_gather(N, V, D, info.num_cores, info.num_subcores)(
        idx2d, embeddings
    )
    return out.reshape(B, H, D)
